# baseline (speedup 1.0000x reference)
import functools

import jax
import jax.numpy as jnp
from jax import lax
from jax.experimental import pallas as pl
from jax.experimental.pallas import tpu as pltpu
from jax.experimental.pallas import tpu_sc as plsc

N = 10000
E = 320000
HID = 128
NCLS = 64
BN_EPS = 1e-5

NC = 2
NS = 16
NW = NC * NS
CHUNK = 128
TOTCHUNK = E // CHUNK
ITERS = 80
RN = 2
IDN = 8
RPT = 624
TAIL0 = NS * RPT
TAIL = N - TAIL0

SELU_ALPHA = 1.6732632423543772
SELU_SCALE = 1.0507009873554805


def _selu(z):
    return SELU_SCALE * jnp.where(z > 0, z, SELU_ALPHA * (jnp.exp(z) - 1.0))


def _make_agg(D):
    mesh = plsc.VectorSubcoreMesh(core_axis_name="c", subcore_axis_name="s")

    @functools.partial(
        pl.kernel,
        out_type=jax.ShapeDtypeStruct((NC, N, D), jnp.float32),
        mesh=mesh,
        scratch_types=[
            pltpu.VMEM((IDN, 2, CHUNK), jnp.int32),
            pltpu.VMEM((RN, CHUNK, D), jnp.float32),
            pltpu.VMEM_SHARED((N, D), jnp.float32),
        ] + [pltpu.SemaphoreType.DMA] * (RN + IDN),
    )
    def agg(h_hbm, ei_hbm, zeros_hbm, out_hbm,
            ids_v, rows_v, acc, *sems):
        gsem = sems[:RN]
        isem = sems[RN:]
        c = lax.axis_index("c")
        s = lax.axis_index("s")
        wid = c * NS + s
        row0 = s * RPT

        def _valid(j):
            return wid + NW * j < TOTCHUNK

        def _eoff(j):
            return pl.multiple_of((wid + NW * j) * CHUNK, 128)

        def id_fetch(j, slot):
            off = _eoff(j)
            pltpu.async_copy(ei_hbm.at[:, pl.ds(off, CHUNK)], ids_v.at[slot],
                             isem[slot])

        def id_wait(j, slot):
            off = _eoff(j)
            pltpu.make_async_copy(ei_hbm.at[:, pl.ds(off, CHUNK)],
                                  ids_v.at[slot], isem[slot]).wait()

        def gather_start(j, slot, idslot):
            pltpu.async_copy(h_hbm.at[ids_v.at[idslot, 0]], rows_v.at[slot],
                             gsem[slot])

        def gather_wait(j, slot, idslot):
            pltpu.make_async_copy(h_hbm.at[ids_v.at[idslot, 0]],
                                  rows_v.at[slot], gsem[slot]).wait()

        def _init(src):
            pltpu.sync_copy(src.at[pl.ds(row0, RPT)], acc.at[pl.ds(row0, RPT)])

            @pl.when(s == NS - 1)
            def _tail():
                pltpu.sync_copy(src.at[pl.ds(TAIL0, TAIL)],
                                acc.at[pl.ds(TAIL0, TAIL)])

        for bb in range(IDN):
            id_fetch(bb, bb)

        @pl.when(c == 0)
        def _initx():
            _init(h_hbm)

        @pl.when(c == 1)
        def _initz():
            _init(zeros_hbm)

        for b in range(RN):
            id_wait(b, b)
            gather_start(b, b, b)
        plsc.subcore_barrier()

        def body(jo, carry):
            for q in range(IDN):
                j = jo * IDN + q
                b = q % RN

                @pl.when(_valid(j))
                def _consume():
                    gather_wait(j, b, q)
                    pltpu.sync_copy(rows_v.at[b], acc.at[ids_v.at[q, 1]],
                                    add=True)

                @pl.when(_valid(j + IDN))
                def _idrefill():
                    id_fetch(j + IDN, q)

                @pl.when(_valid(j + RN))
                def _refill():
                    nq = (q + RN) % IDN
                    id_wait(j + RN, nq)
                    gather_start(j + RN, b, nq)
            return carry

        lax.fori_loop(0, ITERS // IDN, body, 0)
        plsc.subcore_barrier()
        pltpu.sync_copy(acc.at[pl.ds(row0, RPT)],
                        out_hbm.at[c].at[pl.ds(row0, RPT)])

        @pl.when(s == NS - 1)
        def _():
            pltpu.sync_copy(acc.at[pl.ds(TAIL0, TAIL)],
                            out_hbm.at[c].at[pl.ds(TAIL0, TAIL)])

    return agg


def _dense1(p, W1a, b1a, W1b, b1b, g1, be1):
    def body(p_ref, wa, ba, wb, bb, gg, bb2, out_ref):
        z = p_ref[0] + p_ref[1]
        z = jnp.dot(z, wa[...], preferred_element_type=jnp.float32) + ba[...]
        z = jnp.maximum(z, 0.0)
        z = jnp.dot(z, wb[...], preferred_element_type=jnp.float32) + bb[...]
        h = _selu(z)
        mean = jnp.mean(h, axis=0, keepdims=True)
        var = jnp.mean((h - mean) ** 2, axis=0, keepdims=True)
        out_ref[...] = gg[...] * (h - mean) * lax.rsqrt(var + BN_EPS) + bb2[...]

    return pl.pallas_call(
        body,
        out_shape=jax.ShapeDtypeStruct((N, HID), jnp.float32),
    )(p, W1a, b1a, W1b, b1b, g1, be1)


def _dense2(p, W2a, b2a, W2b, b2b, g2, be2):
    def body(p_ref, wa, ba, wb, bb, gg, bb2, out_ref):
        z = p_ref[0] + p_ref[1]
        z = jnp.dot(z, wa[...], preferred_element_type=jnp.float32) + ba[...]
        z = jnp.maximum(z, 0.0)
        z = jnp.dot(z, wb[...], preferred_element_type=jnp.float32) + bb[...]
        h2 = _selu(z)
        mean = jnp.mean(h2, axis=0, keepdims=True)
        var = jnp.mean((h2 - mean) ** 2, axis=0, keepdims=True)
        h2 = gg[...] * (h2 - mean) * lax.rsqrt(var + BN_EPS) + bb2[...]
        m = jnp.max(h2, axis=1, keepdims=True)
        e = jnp.exp(h2 - m)
        out_ref[...] = e / jnp.sum(e, axis=1, keepdims=True)

    return pl.pallas_call(
        body,
        out_shape=jax.ShapeDtypeStruct((N, NCLS), jnp.float32),
    )(p, W2a, b2a, W2b, b2b, g2, be2)


def kernel(x, edge_index, W1a, b1a, W1b, b1b, bn1_g, bn1_b,
           W2a, b2a, W2b, b2b, bn2_g, bn2_b,
           g, A_k, D, Kindices, de, M, I):
    ei = edge_index.astype(jnp.int32)
    zeros = jnp.zeros((N, HID), dtype=jnp.float32)

    agg = _make_agg(HID)

    b1a_ = b1a.reshape(1, HID)
    b1b_ = b1b.reshape(1, HID)
    g1_ = bn1_g.reshape(1, HID)
    be1_ = bn1_b.reshape(1, HID)
    b2a_ = b2a.reshape(1, HID)
    b2b_ = b2b.reshape(1, NCLS)
    g2_ = bn2_g.reshape(1, NCLS)
    be2_ = bn2_b.reshape(1, NCLS)

    p = agg(x, ei, zeros)
    h = _dense1(p, W1a, b1a_, W1b, b1b_, g1_, be1_)
    p2 = agg(h, ei, zeros)
    out = _dense2(p2, W2a, b2a_, W2b, b2b_, g2_, be2_)
    return out


# speedup vs baseline: 1.0627x; 1.0627x over previous
import functools

import jax
import jax.numpy as jnp
from jax import lax
from jax.experimental import pallas as pl
from jax.experimental.pallas import tpu as pltpu
from jax.experimental.pallas import tpu_sc as plsc

N = 10000
E = 320000
HID = 128
NCLS = 64
BN_EPS = 1e-5

NC = 2
NS = 16
NW = NC * NS
EPW = E // NW
CHUNK = 40
NCHUNK = EPW // CHUNK
RN = 5
NBUF = 4
IDN = 10
RPT = 624
TAIL0 = NS * RPT
TAIL = N - TAIL0

SELU_ALPHA = 1.6732632423543772
SELU_SCALE = 1.0507009873554805


def _selu(z):
    return SELU_SCALE * jnp.where(z > 0, z, SELU_ALPHA * (jnp.exp(z) - 1.0))


def _make_agg(D):
    mesh = plsc.VectorSubcoreMesh(core_axis_name="c", subcore_axis_name="s")

    @functools.partial(
        pl.kernel,
        out_type=jax.ShapeDtypeStruct((NC, N, D), jnp.float32),
        mesh=mesh,
        scratch_types=[
            pltpu.VMEM((IDN, 2, CHUNK), jnp.int32),
            pltpu.VMEM((RN, CHUNK, D), jnp.float32),
            pltpu.VMEM_SHARED((N, D), jnp.float32),
        ] + [pltpu.SemaphoreType.DMA] * (2 * RN + IDN),
    )
    def agg(h_hbm, srcf_hbm, dstf_hbm, zeros_hbm, out_hbm,
            ids_v, rows_v, acc, *sems):
        gsem = sems[:RN]
        ssem = sems[RN:2 * RN]
        isem = sems[2 * RN:]
        c = lax.axis_index("c")
        s = lax.axis_index("s")
        wid = c * NS + s
        row0 = s * RPT

        def _eoff(j):
            return pl.multiple_of(wid * EPW + j * CHUNK, 8)

        def id_fetch(j, slot):
            off = _eoff(j)
            pltpu.async_copy(srcf_hbm.at[pl.ds(off, CHUNK)], ids_v.at[slot, 0],
                             isem[slot])
            pltpu.async_copy(dstf_hbm.at[pl.ds(off, CHUNK)], ids_v.at[slot, 1],
                             isem[slot])

        def id_wait(j, slot):
            off = _eoff(j)
            pltpu.make_async_copy(srcf_hbm.at[pl.ds(off, CHUNK)],
                                  ids_v.at[slot, 0], isem[slot]).wait()
            pltpu.make_async_copy(dstf_hbm.at[pl.ds(off, CHUNK)],
                                  ids_v.at[slot, 1], isem[slot]).wait()

        def gather_start(j, slot, idslot):
            pltpu.async_copy(h_hbm.at[ids_v.at[idslot, 0]], rows_v.at[slot],
                             gsem[slot])

        def gather_wait(j, slot, idslot):
            pltpu.make_async_copy(h_hbm.at[ids_v.at[idslot, 0]],
                                  rows_v.at[slot], gsem[slot]).wait()

        def scatter_start(j, slot, idslot):
            pltpu.async_copy(rows_v.at[slot], acc.at[ids_v.at[idslot, 1]],
                             ssem[slot], add=True)

        def scatter_wait(j, slot, idslot):
            pltpu.make_async_copy(rows_v.at[slot], acc.at[ids_v.at[idslot, 1]],
                                  ssem[slot]).wait()

        def _init(src):
            pltpu.sync_copy(src.at[pl.ds(row0, RPT)], acc.at[pl.ds(row0, RPT)])

            @pl.when(s == NS - 1)
            def _tail():
                pltpu.sync_copy(src.at[pl.ds(TAIL0, TAIL)],
                                acc.at[pl.ds(TAIL0, TAIL)])

        for bb in range(IDN):
            id_fetch(bb, bb)

        @pl.when(c == 0)
        def _initx():
            _init(h_hbm)

        @pl.when(c == 1)
        def _initz():
            _init(zeros_hbm)

        for b in range(NBUF):
            id_wait(b, b)
            gather_start(b, b, b)
        plsc.subcore_barrier()

        def body(jo, carry):
            for bb in range(IDN):
                j = jo * IDN + bb
                b = bb % RN
                gather_wait(j, b, bb)
                scatter_start(j, b, bb)

                @pl.when((j >= 1) & (j + NBUF < NCHUNK))
                def _drain_prev():
                    scatter_wait(j - 1, (b + RN - 1) % RN, (bb + IDN - 1) % IDN)

                @pl.when((j >= 1) & (j + IDN - 1 < NCHUNK))
                def _idrefill():
                    id_fetch(j + IDN - 1, (bb + IDN - 1) % IDN)

                @pl.when(j + NBUF < NCHUNK)
                def _refill():
                    nb = (bb + NBUF) % IDN
                    id_wait(j + NBUF, nb)
                    gather_start(j + NBUF, (b + NBUF) % RN, nb)
            return carry

        lax.fori_loop(0, NCHUNK // IDN, body, 0)
        for m in range(NCHUNK - RN, NCHUNK):
            scatter_wait(m, m % RN, m % IDN)
        plsc.subcore_barrier()
        pltpu.sync_copy(acc.at[pl.ds(row0, RPT)],
                        out_hbm.at[c].at[pl.ds(row0, RPT)])

        @pl.when(s == NS - 1)
        def _():
            pltpu.sync_copy(acc.at[pl.ds(TAIL0, TAIL)],
                            out_hbm.at[c].at[pl.ds(TAIL0, TAIL)])

    return agg


def _dense1(p, W1a, b1a, W1b, b1b, g1, be1):
    def body(p_ref, wa, ba, wb, bb, gg, bb2, out_ref):
        z = p_ref[0] + p_ref[1]
        z = jnp.dot(z, wa[...], preferred_element_type=jnp.float32) + ba[...]
        z = jnp.maximum(z, 0.0)
        z = jnp.dot(z, wb[...], preferred_element_type=jnp.float32) + bb[...]
        h = _selu(z)
        mean = jnp.mean(h, axis=0, keepdims=True)
        var = jnp.mean((h - mean) ** 2, axis=0, keepdims=True)
        out_ref[...] = gg[...] * (h - mean) * lax.rsqrt(var + BN_EPS) + bb2[...]

    return pl.pallas_call(
        body,
        out_shape=jax.ShapeDtypeStruct((N, HID), jnp.float32),
    )(p, W1a, b1a, W1b, b1b, g1, be1)


def _dense2(p, W2a, b2a, W2b, b2b, g2, be2):
    def body(p_ref, wa, ba, wb, bb, gg, bb2, out_ref):
        z = p_ref[0] + p_ref[1]
        z = jnp.dot(z, wa[...], preferred_element_type=jnp.float32) + ba[...]
        z = jnp.maximum(z, 0.0)
        z = jnp.dot(z, wb[...], preferred_element_type=jnp.float32) + bb[...]
        h2 = _selu(z)
        mean = jnp.mean(h2, axis=0, keepdims=True)
        var = jnp.mean((h2 - mean) ** 2, axis=0, keepdims=True)
        h2 = gg[...] * (h2 - mean) * lax.rsqrt(var + BN_EPS) + bb2[...]
        m = jnp.max(h2, axis=1, keepdims=True)
        e = jnp.exp(h2 - m)
        out_ref[...] = e / jnp.sum(e, axis=1, keepdims=True)

    return pl.pallas_call(
        body,
        out_shape=jax.ShapeDtypeStruct((N, NCLS), jnp.float32),
    )(p, W2a, b2a, W2b, b2b, g2, be2)


def _repack_ids(ei):
    def body(ei_ref, s_ref, d_ref):
        s_ref[...] = ei_ref[0]
        d_ref[...] = ei_ref[1]

    return pl.pallas_call(
        body,
        out_shape=(jax.ShapeDtypeStruct((E,), jnp.int32),
                   jax.ShapeDtypeStruct((E,), jnp.int32)),
    )(ei)


def kernel(x, edge_index, W1a, b1a, W1b, b1b, bn1_g, bn1_b,
           W2a, b2a, W2b, b2b, bn2_g, bn2_b,
           g, A_k, D, Kindices, de, M, I):
    ei = edge_index.astype(jnp.int32)
    srcf, dstf = _repack_ids(ei)
    zeros = jnp.zeros((N, HID), dtype=jnp.float32)

    agg = _make_agg(HID)

    b1a_ = b1a.reshape(1, HID)
    b1b_ = b1b.reshape(1, HID)
    g1_ = bn1_g.reshape(1, HID)
    be1_ = bn1_b.reshape(1, HID)
    b2a_ = b2a.reshape(1, HID)
    b2b_ = b2b.reshape(1, NCLS)
    g2_ = bn2_g.reshape(1, NCLS)
    be2_ = bn2_b.reshape(1, NCLS)

    p = agg(x, srcf, dstf, zeros)
    h = _dense1(p, W1a, b1a_, W1b, b1b_, g1_, be1_)
    p2 = agg(h, srcf, dstf, zeros)
    out = _dense2(p2, W2a, b2a_, W2b, b2b_, g2_, be2_)
    return out


# speedup vs baseline: 1.0895x; 1.0252x over previous
import functools

import jax
import jax.numpy as jnp
from jax import lax
from jax.experimental import pallas as pl
from jax.experimental.pallas import tpu as pltpu
from jax.experimental.pallas import tpu_sc as plsc

N = 10000
E = 320000
HID = 128
NCLS = 64
BN_EPS = 1e-5

NC = 2
NS = 16
NW = NC * NS
EPW = E // NW
CHUNK = 40
NCHUNK = EPW // CHUNK
RN = 5
NBUF = 4
IDN = 10
RPT = 624
TAIL0 = NS * RPT
TAIL = N - TAIL0

SELU_ALPHA = 1.6732632423543772
SELU_SCALE = 1.0507009873554805


def _selu(z):
    return SELU_SCALE * jnp.where(z > 0, z, SELU_ALPHA * (jnp.exp(z) - 1.0))


def _make_agg(D):
    mesh = plsc.VectorSubcoreMesh(core_axis_name="c", subcore_axis_name="s")

    @functools.partial(
        pl.kernel,
        out_type=jax.ShapeDtypeStruct((NC, N, D), jnp.float32),
        mesh=mesh,
        scratch_types=[
            pltpu.VMEM((IDN, 2, CHUNK), jnp.int32),
            pltpu.VMEM((RN, CHUNK, D), jnp.float32),
            pltpu.VMEM_SHARED((N, D), jnp.float32),
        ] + [pltpu.SemaphoreType.DMA] * (2 * RN + IDN),
    )
    def agg(h_hbm, srcf_hbm, dstf_hbm, out_hbm,
            ids_v, rows_v, acc, *sems):
        gsem = sems[:RN]
        ssem = sems[RN:2 * RN]
        isem = sems[2 * RN:]
        c = lax.axis_index("c")
        s = lax.axis_index("s")
        wid = c * NS + s
        row0 = s * RPT

        def _eoff(j):
            return pl.multiple_of(wid * EPW + j * CHUNK, 8)

        def id_fetch(j, slot):
            off = _eoff(j)
            pltpu.async_copy(srcf_hbm.at[pl.ds(off, CHUNK)], ids_v.at[slot, 0],
                             isem[slot])
            pltpu.async_copy(dstf_hbm.at[pl.ds(off, CHUNK)], ids_v.at[slot, 1],
                             isem[slot])

        def id_wait(j, slot):
            off = _eoff(j)
            pltpu.make_async_copy(srcf_hbm.at[pl.ds(off, CHUNK)],
                                  ids_v.at[slot, 0], isem[slot]).wait()
            pltpu.make_async_copy(dstf_hbm.at[pl.ds(off, CHUNK)],
                                  ids_v.at[slot, 1], isem[slot]).wait()

        def gather_start(j, slot, idslot):
            pltpu.async_copy(h_hbm.at[ids_v.at[idslot, 0]], rows_v.at[slot],
                             gsem[slot])

        def gather_wait(j, slot, idslot):
            pltpu.make_async_copy(h_hbm.at[ids_v.at[idslot, 0]],
                                  rows_v.at[slot], gsem[slot]).wait()

        def scatter_start(j, slot, idslot):
            pltpu.async_copy(rows_v.at[slot], acc.at[ids_v.at[idslot, 1]],
                             ssem[slot], add=True)

        def scatter_wait(j, slot, idslot):
            pltpu.make_async_copy(rows_v.at[slot], acc.at[ids_v.at[idslot, 1]],
                                  ssem[slot]).wait()

        for bb in range(IDN):
            id_fetch(bb, bb)

        use_h = (s < NS // 2) == (c == 0)

        @pl.when(use_h)
        def _inith():
            pltpu.sync_copy(h_hbm.at[pl.ds(row0, RPT)],
                            acc.at[pl.ds(row0, RPT)])

            @pl.when(s == NS - 1)
            def _tailh():
                pltpu.sync_copy(h_hbm.at[pl.ds(TAIL0, TAIL)],
                                acc.at[pl.ds(TAIL0, TAIL)])

        @pl.when(jnp.logical_not(use_h))
        def _initz():
            z16 = jnp.zeros((16,), jnp.float32)

            def zrow(r, carry):
                for k in range(D // 16):
                    rows_v[0, r, pl.ds(k * 16, 16)] = z16
                return carry

            lax.fori_loop(0, CHUNK, zrow, 0)
            for k in range(RPT // CHUNK):
                pltpu.sync_copy(rows_v.at[0],
                                acc.at[pl.ds(row0 + k * CHUNK, CHUNK)])
            rem = RPT % CHUNK
            if rem:
                pltpu.sync_copy(rows_v.at[0, pl.ds(0, rem)],
                                acc.at[pl.ds(row0 + RPT - rem, rem)])

            @pl.when(s == NS - 1)
            def _tailz():
                pltpu.sync_copy(rows_v.at[0, pl.ds(0, TAIL)],
                                acc.at[pl.ds(TAIL0, TAIL)])

        for b in range(NBUF):
            id_wait(b, b)
            gather_start(b, b, b)
        plsc.subcore_barrier()

        def body(jo, carry):
            for bb in range(IDN):
                j = jo * IDN + bb
                b = bb % RN
                gather_wait(j, b, bb)
                scatter_start(j, b, bb)

                @pl.when((j >= 1) & (j + NBUF < NCHUNK))
                def _drain_prev():
                    scatter_wait(j - 1, (b + RN - 1) % RN, (bb + IDN - 1) % IDN)

                @pl.when((j >= 1) & (j + IDN - 1 < NCHUNK))
                def _idrefill():
                    id_fetch(j + IDN - 1, (bb + IDN - 1) % IDN)

                @pl.when(j + NBUF < NCHUNK)
                def _refill():
                    nb = (bb + NBUF) % IDN
                    id_wait(j + NBUF, nb)
                    gather_start(j + NBUF, (b + NBUF) % RN, nb)
            return carry

        lax.fori_loop(0, NCHUNK // IDN, body, 0)
        for m in range(NCHUNK - RN, NCHUNK):
            scatter_wait(m, m % RN, m % IDN)
        plsc.subcore_barrier()
        pltpu.sync_copy(acc.at[pl.ds(row0, RPT)],
                        out_hbm.at[c].at[pl.ds(row0, RPT)])

        @pl.when(s == NS - 1)
        def _():
            pltpu.sync_copy(acc.at[pl.ds(TAIL0, TAIL)],
                            out_hbm.at[c].at[pl.ds(TAIL0, TAIL)])

    return agg


def _dense1(p, W1a, b1a, W1b, b1b, g1, be1):
    def body(p_ref, wa, ba, wb, bb, gg, bb2, out_ref):
        z = p_ref[0] + p_ref[1]
        z = jnp.dot(z, wa[...], preferred_element_type=jnp.float32) + ba[...]
        z = jnp.maximum(z, 0.0)
        z = jnp.dot(z, wb[...], preferred_element_type=jnp.float32) + bb[...]
        h = _selu(z)
        mean = jnp.mean(h, axis=0, keepdims=True)
        var = jnp.mean((h - mean) ** 2, axis=0, keepdims=True)
        out_ref[...] = gg[...] * (h - mean) * lax.rsqrt(var + BN_EPS) + bb2[...]

    return pl.pallas_call(
        body,
        out_shape=jax.ShapeDtypeStruct((N, HID), jnp.float32),
    )(p, W1a, b1a, W1b, b1b, g1, be1)


def _dense2(p, W2a, b2a, W2b, b2b, g2, be2):
    def body(p_ref, wa, ba, wb, bb, gg, bb2, out_ref):
        z = p_ref[0] + p_ref[1]
        z = jnp.dot(z, wa[...], preferred_element_type=jnp.float32) + ba[...]
        z = jnp.maximum(z, 0.0)
        z = jnp.dot(z, wb[...], preferred_element_type=jnp.float32) + bb[...]
        h2 = _selu(z)
        mean = jnp.mean(h2, axis=0, keepdims=True)
        var = jnp.mean((h2 - mean) ** 2, axis=0, keepdims=True)
        h2 = gg[...] * (h2 - mean) * lax.rsqrt(var + BN_EPS) + bb2[...]
        m = jnp.max(h2, axis=1, keepdims=True)
        e = jnp.exp(h2 - m)
        out_ref[...] = e / jnp.sum(e, axis=1, keepdims=True)

    return pl.pallas_call(
        body,
        out_shape=jax.ShapeDtypeStruct((N, NCLS), jnp.float32),
    )(p, W2a, b2a, W2b, b2b, g2, be2)


def _repack_ids(ei):
    def body(ei_ref, s_ref, d_ref):
        s_ref[...] = ei_ref[0]
        d_ref[...] = ei_ref[1]

    return pl.pallas_call(
        body,
        out_shape=(jax.ShapeDtypeStruct((E,), jnp.int32),
                   jax.ShapeDtypeStruct((E,), jnp.int32)),
    )(ei)


def kernel(x, edge_index, W1a, b1a, W1b, b1b, bn1_g, bn1_b,
           W2a, b2a, W2b, b2b, bn2_g, bn2_b,
           g, A_k, D, Kindices, de, M, I):
    ei = edge_index.astype(jnp.int32)
    srcf, dstf = _repack_ids(ei)

    agg = _make_agg(HID)

    b1a_ = b1a.reshape(1, HID)
    b1b_ = b1b.reshape(1, HID)
    g1_ = bn1_g.reshape(1, HID)
    be1_ = bn1_b.reshape(1, HID)
    b2a_ = b2a.reshape(1, HID)
    b2b_ = b2b.reshape(1, NCLS)
    g2_ = bn2_g.reshape(1, NCLS)
    be2_ = bn2_b.reshape(1, NCLS)

    p = agg(x, srcf, dstf)
    h = _dense1(p, W1a, b1a_, W1b, b1b_, g1_, be1_)
    p2 = agg(h, srcf, dstf)
    out = _dense2(p2, W2a, b2a_, W2b, b2b_, g2_, be2_)
    return out


# speedup vs baseline: 1.1146x; 1.0230x over previous
import functools

import jax
import jax.numpy as jnp
from jax import lax
from jax.experimental import pallas as pl
from jax.experimental.pallas import tpu as pltpu
from jax.experimental.pallas import tpu_sc as plsc

N = 10000
E = 320000
HID = 128
NCLS = 64
BN_EPS = 1e-5

NC = 2
NS = 16
NW = NC * NS
EPW = E // NW
CHUNK = 40
NCHUNK = EPW // CHUNK
RN = 5
NBUF = 4
IDN = 10
RPT = 624
TAIL0 = NS * RPT
TAIL = N - TAIL0

SELU_ALPHA = 1.6732632423543772
SELU_SCALE = 1.0507009873554805


def _selu(z):
    return SELU_SCALE * jnp.where(z > 0, z, SELU_ALPHA * (jnp.exp(z) - 1.0))


def _make_agg(D):
    mesh = plsc.VectorSubcoreMesh(core_axis_name="c", subcore_axis_name="s")

    @functools.partial(
        pl.kernel,
        out_type=jax.ShapeDtypeStruct((NC, N, D), jnp.float32),
        mesh=mesh,
        scratch_types=[
            pltpu.VMEM((IDN, 2, CHUNK), jnp.int32),
            pltpu.VMEM((RN, CHUNK, D), jnp.float32),
            pltpu.VMEM_SHARED((N, D), jnp.float32),
        ] + [pltpu.SemaphoreType.DMA] * (2 * RN + IDN),
    )
    def agg(h_hbm, srcf_hbm, dstf_hbm, out_hbm,
            ids_v, rows_v, acc, *sems):
        gsem = sems[:RN]
        ssem = sems[RN:2 * RN]
        isem = sems[2 * RN:]
        c = lax.axis_index("c")
        s = lax.axis_index("s")
        wid = c * NS + s
        row0 = s * RPT

        def _eoff(j):
            return pl.multiple_of(wid * EPW + j * CHUNK, 8)

        def id_fetch(j, slot):
            off = _eoff(j)
            pltpu.async_copy(srcf_hbm.at[pl.ds(off, CHUNK)], ids_v.at[slot, 0],
                             isem[slot])
            pltpu.async_copy(dstf_hbm.at[pl.ds(off, CHUNK)], ids_v.at[slot, 1],
                             isem[slot])

        def id_wait(j, slot):
            off = _eoff(j)
            pltpu.make_async_copy(srcf_hbm.at[pl.ds(off, CHUNK)],
                                  ids_v.at[slot, 0], isem[slot]).wait()
            pltpu.make_async_copy(dstf_hbm.at[pl.ds(off, CHUNK)],
                                  ids_v.at[slot, 1], isem[slot]).wait()

        def gather_start(j, slot, idslot):
            pltpu.async_copy(h_hbm.at[ids_v.at[idslot, 0]], rows_v.at[slot],
                             gsem[slot])

        def gather_wait(j, slot, idslot):
            pltpu.make_async_copy(h_hbm.at[ids_v.at[idslot, 0]],
                                  rows_v.at[slot], gsem[slot]).wait()

        def scatter_start(j, slot, idslot):
            pltpu.async_copy(rows_v.at[slot], acc.at[ids_v.at[idslot, 1]],
                             ssem[slot], add=True)

        def scatter_wait(j, slot, idslot):
            pltpu.make_async_copy(rows_v.at[slot], acc.at[ids_v.at[idslot, 1]],
                                  ssem[slot]).wait()

        for bb in range(IDN):
            id_fetch(bb, bb)

        use_h = (s < NS // 2) == (c == 0)

        @pl.when(use_h)
        def _inith():
            pltpu.sync_copy(h_hbm.at[pl.ds(row0, RPT)],
                            acc.at[pl.ds(row0, RPT)])

            @pl.when(s == NS - 1)
            def _tailh():
                pltpu.sync_copy(h_hbm.at[pl.ds(TAIL0, TAIL)],
                                acc.at[pl.ds(TAIL0, TAIL)])

        @pl.when(jnp.logical_not(use_h))
        def _initz():
            z16 = jnp.zeros((16,), jnp.float32)

            def zrow(r, carry):
                for k in range(D // 16):
                    rows_v[0, r, pl.ds(k * 16, 16)] = z16
                return carry

            lax.fori_loop(0, CHUNK, zrow, 0)
            for k in range(RPT // CHUNK):
                pltpu.sync_copy(rows_v.at[0],
                                acc.at[pl.ds(row0 + k * CHUNK, CHUNK)])
            rem = RPT % CHUNK
            if rem:
                pltpu.sync_copy(rows_v.at[0, pl.ds(0, rem)],
                                acc.at[pl.ds(row0 + RPT - rem, rem)])

            @pl.when(s == NS - 1)
            def _tailz():
                pltpu.sync_copy(rows_v.at[0, pl.ds(0, TAIL)],
                                acc.at[pl.ds(TAIL0, TAIL)])

        for b in range(NBUF):
            id_wait(b, b)
            gather_start(b, b, b)
        plsc.subcore_barrier()

        def body(jo, carry):
            for bb in range(IDN):
                j = jo * IDN + bb
                b = bb % RN
                gather_wait(j, b, bb)
                scatter_start(j, b, bb)

                @pl.when((j >= 1) & (j + NBUF < NCHUNK))
                def _drain_prev():
                    scatter_wait(j - 1, (b + RN - 1) % RN, (bb + IDN - 1) % IDN)

                @pl.when((j >= 1) & (j + IDN - 1 < NCHUNK))
                def _idrefill():
                    id_fetch(j + IDN - 1, (bb + IDN - 1) % IDN)

                @pl.when(j + NBUF < NCHUNK)
                def _refill():
                    nb = (bb + NBUF) % IDN
                    id_wait(j + NBUF, nb)
                    gather_start(j + NBUF, (b + NBUF) % RN, nb)
            return carry

        lax.fori_loop(0, NCHUNK // IDN, body, 0)
        for m in range(NCHUNK - RN, NCHUNK):
            scatter_wait(m, m % RN, m % IDN)
        plsc.subcore_barrier()
        pltpu.sync_copy(acc.at[pl.ds(row0, RPT)],
                        out_hbm.at[c].at[pl.ds(row0, RPT)])

        @pl.when(s == NS - 1)
        def _():
            pltpu.sync_copy(acc.at[pl.ds(TAIL0, TAIL)],
                            out_hbm.at[c].at[pl.ds(TAIL0, TAIL)])

    return agg


def _dense1(p, W1a, b1a, W1b, b1b, g1, be1):
    def body(p_ref, wa, ba, wb, bb, gg, bb2, out_ref):
        z = p_ref[0] + p_ref[1]
        z = jnp.dot(z, wa[...], preferred_element_type=jnp.float32) + ba[...]
        z = jnp.maximum(z, 0.0)
        z = jnp.dot(z, wb[...], preferred_element_type=jnp.float32) + bb[...]
        h = _selu(z)
        mean = jnp.mean(h, axis=0, keepdims=True)
        var = jnp.mean((h - mean) ** 2, axis=0, keepdims=True)
        out_ref[...] = gg[...] * (h - mean) * lax.rsqrt(var + BN_EPS) + bb2[...]

    return pl.pallas_call(
        body,
        out_shape=jax.ShapeDtypeStruct((N, HID), jnp.float32),
    )(p, W1a, b1a, W1b, b1b, g1, be1)


def _dense2(p, W2a, b2a, W2b, b2b, g2, be2):
    def body(p_ref, wa, ba, wb, bb, gg, bb2, out_ref):
        z = p_ref[0] + p_ref[1]
        z = jnp.dot(z, wa[...], preferred_element_type=jnp.float32) + ba[...]
        z = jnp.maximum(z, 0.0)
        z = jnp.dot(z, wb[...], preferred_element_type=jnp.float32) + bb[...]
        h2 = _selu(z)
        mean = jnp.mean(h2, axis=0, keepdims=True)
        var = jnp.mean((h2 - mean) ** 2, axis=0, keepdims=True)
        h2 = gg[...] * (h2 - mean) * lax.rsqrt(var + BN_EPS) + bb2[...]
        m = jnp.max(h2, axis=1, keepdims=True)
        e = jnp.exp(h2 - m)
        sm = e / jnp.sum(e, axis=1, keepdims=True)
        out_ref[...] = sm.T

    return pl.pallas_call(
        body,
        out_shape=jax.ShapeDtypeStruct((NCLS, N), jnp.float32),
    )(p, W2a, b2a, W2b, b2b, g2, be2)


def _repack_ids(ei):
    def body(ei_ref, s_ref, d_ref):
        s_ref[...] = ei_ref[0]
        d_ref[...] = ei_ref[1]

    return pl.pallas_call(
        body,
        out_shape=(jax.ShapeDtypeStruct((E,), jnp.int32),
                   jax.ShapeDtypeStruct((E,), jnp.int32)),
    )(ei)


def kernel(x, edge_index, W1a, b1a, W1b, b1b, bn1_g, bn1_b,
           W2a, b2a, W2b, b2b, bn2_g, bn2_b,
           g, A_k, D, Kindices, de, M, I):
    ei = edge_index.astype(jnp.int32)
    srcf, dstf = _repack_ids(ei)

    agg = _make_agg(HID)

    b1a_ = b1a.reshape(1, HID)
    b1b_ = b1b.reshape(1, HID)
    g1_ = bn1_g.reshape(1, HID)
    be1_ = bn1_b.reshape(1, HID)
    b2a_ = b2a.reshape(1, HID)
    b2b_ = b2b.reshape(1, NCLS)
    g2_ = bn2_g.reshape(1, NCLS)
    be2_ = bn2_b.reshape(1, NCLS)

    p = agg(x, srcf, dstf)
    h = _dense1(p, W1a, b1a_, W1b, b1b_, g1_, be1_)
    p2 = agg(h, srcf, dstf)
    out = _dense2(p2, W2a, b2a_, W2b, b2b_, g2_, be2_)
    return out.T
